# Initial kernel scaffold; baseline (speedup 1.0000x reference)
import functools

import jax
import jax.numpy as jnp
from jax import lax
from jax.experimental import pallas as pl
from jax.experimental.pallas import tpu as pltpu
from jax.experimental.pallas import tpu_sc as plsc

N = 10000
E = 320000
D = 128
EPS = 1e-7

TILES = 16
EPT = E // TILES
CHUNK = 80
NCHUNK = EPT // CHUNK
RPT = N // TILES
RCHUNK = 25
NRCH = RPT // RCHUNK


def _norm(x):
    return jnp.maximum(jnp.sqrt(jnp.sum(x * x, axis=-1, keepdims=True)), EPS)


def _artanh(x):
    x = jnp.clip(x, -1.0 + 1e-6, 1.0 - 1e-6)
    return 0.5 * jnp.log((1.0 + x) / (1.0 - x))


def _pre_body(lx_ref, o_ref):
    x = lx_ref[...]
    n = _norm(x)
    o_ref[...] = _artanh(n) * x / n


def _post_body(eagg_ref, tagg_ref, ex_ref, lx_ref, eo_ref, lo_ref):
    ex = ex_ref[...]
    lx = lx_ref[...]
    eo_ref[...] = 0.5 * eagg_ref[...] + 0.5 * ex

    t = tagg_ref[...]
    nt = _norm(t)
    lorentz_out = jnp.tanh(nt) * t / nt

    def mobius_scalar_mul(r, x):
        n = _norm(x)
        return jnp.tanh(r * _artanh(n)) * x / n

    l_skip = mobius_scalar_mul(0.5, lx)
    l_out = mobius_scalar_mul(0.5, lorentz_out)

    xy = jnp.sum(l_out * l_skip, axis=-1, keepdims=True)
    x2 = jnp.sum(l_out * l_out, axis=-1, keepdims=True)
    y2 = jnp.sum(l_skip * l_skip, axis=-1, keepdims=True)
    num = (1.0 + 2.0 * xy + y2) * l_out + (1.0 - x2) * l_skip
    den = jnp.maximum(1.0 + 2.0 * xy + x2 * y2, EPS)
    lo_ref[...] = num / den


_BR = 1000


def _pre(lx):
    return pl.pallas_call(
        _pre_body,
        out_shape=jax.ShapeDtypeStruct((N, D), jnp.float32),
        grid=(N // _BR,),
        in_specs=[pl.BlockSpec((_BR, D), lambda i: (i, 0))],
        out_specs=pl.BlockSpec((_BR, D), lambda i: (i, 0)),
    )(lx)


def _post(e_agg, t_agg, ex, lx):
    spec = pl.BlockSpec((_BR, D), lambda i: (i, 0))
    return pl.pallas_call(
        _post_body,
        out_shape=(
            jax.ShapeDtypeStruct((N, D), jnp.float32),
            jax.ShapeDtypeStruct((N, D), jnp.float32),
        ),
        grid=(N // _BR,),
        in_specs=[spec, spec, spec, spec],
        out_specs=(spec, spec),
    )(e_agg, t_agg, ex, lx)


def _sc_body(ex_hbm, tx_hbm, src_hbm, dst_hbm, vals_hbm,
             eo_hbm, to_hbm,
             sidx, didx, vvals, rows, tmp, acc, sem):
    c = lax.axis_index("c")
    s = lax.axis_index("s")

    z = jnp.zeros((16,), jnp.float32)
    for i in range(RCHUNK):
        for j in range(8):
            tmp[i, pl.ds(j * 16, 16)] = z

    def zbody(k, carry):
        pltpu.sync_copy(tmp, acc.at[pl.ds(s * RPT + k * RCHUNK, RCHUNK)])
        return carry

    lax.fori_loop(0, NRCH, zbody, 0)
    plsc.subcore_barrier()

    def run(x_hbm, out_hbm):
        def chunk_body(i, carry):
            base = pl.multiple_of(s * EPT + i * CHUNK, 8)
            pltpu.sync_copy(src_hbm.at[pl.ds(base, CHUNK)], sidx)
            pltpu.sync_copy(dst_hbm.at[pl.ds(base, CHUNK)], didx)
            pltpu.sync_copy(vals_hbm.at[pl.ds(base, CHUNK)], vvals)
            pltpu.async_copy(x_hbm.at[sidx], rows, sem).wait()

            def scale(e, carry2):
                vb = plsc.load_gather(vvals, [jnp.full((16,), e, jnp.int32)])
                for j in range(8):
                    sl = pl.ds(j * 16, 16)
                    rows[e, sl] = rows[e, sl] * vb
                return carry2

            lax.fori_loop(0, CHUNK, scale, 0)
            pltpu.sync_copy(rows, acc.at[didx], add=True)
            return carry

        lax.fori_loop(0, NCHUNK, chunk_body, 0)
        plsc.subcore_barrier()

        def rbody(k, carry):
            r0 = s * RPT + k * RCHUNK
            pltpu.sync_copy(acc.at[pl.ds(r0, RCHUNK)], tmp)
            pltpu.sync_copy(tmp, out_hbm.at[pl.ds(r0, RCHUNK)])
            return carry

        lax.fori_loop(0, NRCH, rbody, 0)

    @pl.when(c == 0)
    def _():
        run(ex_hbm, eo_hbm)

    @pl.when(c == 1)
    def _():
        run(tx_hbm, to_hbm)


@functools.partial(
    pl.kernel,
    out_type=(
        jax.ShapeDtypeStruct((N, D), jnp.float32),
        jax.ShapeDtypeStruct((N, D), jnp.float32),
    ),
    mesh=plsc.VectorSubcoreMesh(core_axis_name="c", subcore_axis_name="s"),
    scratch_types=[
        pltpu.VMEM((CHUNK,), jnp.int32),
        pltpu.VMEM((CHUNK,), jnp.int32),
        pltpu.VMEM((CHUNK,), jnp.float32),
        pltpu.VMEM((CHUNK, D), jnp.float32),
        pltpu.VMEM((RCHUNK, D), jnp.float32),
        pltpu.VMEM_SHARED((N, D), jnp.float32),
        pltpu.SemaphoreType.DMA,
    ],
)
def _sc_spmm(*refs):
    _sc_body(*refs)


def kernel(euclidean_x, lorentz_x, adj_indices, adj_values):
    src = adj_indices[0]
    dst = adj_indices[1]
    tangent_x = _pre(lorentz_x)
    e_agg, t_agg = _sc_spmm(euclidean_x, tangent_x, src, dst, adj_values)
    return _post(e_agg, t_agg, euclidean_x, lorentz_x)


# speedup vs baseline: 3.8780x; 3.8780x over previous
import functools

import jax
import jax.numpy as jnp
from jax import lax
from jax.experimental import pallas as pl
from jax.experimental.pallas import tpu as pltpu
from jax.experimental.pallas import tpu_sc as plsc

N = 10000
E = 320000
D = 128
EPS = 1e-7

TILES = 16
EPT = E // TILES
CHUNK = 80
NCHUNK = EPT // CHUNK
RCHUNK = 40
NRBLK = N // RCHUNK
NRIT = -(-NRBLK // TILES)


def _norm(x):
    return jnp.maximum(jnp.sqrt(jnp.sum(x * x, axis=-1, keepdims=True)), EPS)


def _artanh(x):
    x = jnp.clip(x, -1.0 + 1e-6, 1.0 - 1e-6)
    return 0.5 * jnp.log((1.0 + x) / (1.0 - x))


def _pre_body(lx_ref, o_ref):
    x = lx_ref[...]
    n = _norm(x)
    o_ref[...] = _artanh(n) * x / n


def _post_body(eagg_ref, tagg_ref, ex_ref, lx_ref, eo_ref, lo_ref):
    ex = ex_ref[...]
    lx = lx_ref[...]
    eo_ref[...] = 0.5 * eagg_ref[...] + 0.5 * ex

    t = tagg_ref[...]
    nt = _norm(t)
    lorentz_out = jnp.tanh(nt) * t / nt

    def mobius_scalar_mul(r, x):
        n = _norm(x)
        return jnp.tanh(r * _artanh(n)) * x / n

    l_skip = mobius_scalar_mul(0.5, lx)
    l_out = mobius_scalar_mul(0.5, lorentz_out)

    xy = jnp.sum(l_out * l_skip, axis=-1, keepdims=True)
    x2 = jnp.sum(l_out * l_out, axis=-1, keepdims=True)
    y2 = jnp.sum(l_skip * l_skip, axis=-1, keepdims=True)
    num = (1.0 + 2.0 * xy + y2) * l_out + (1.0 - x2) * l_skip
    den = jnp.maximum(1.0 + 2.0 * xy + x2 * y2, EPS)
    lo_ref[...] = num / den


_BR = 1000


def _pre(lx):
    return pl.pallas_call(
        _pre_body,
        out_shape=jax.ShapeDtypeStruct((N, D), jnp.float32),
        grid=(N // _BR,),
        in_specs=[pl.BlockSpec((_BR, D), lambda i: (i, 0))],
        out_specs=pl.BlockSpec((_BR, D), lambda i: (i, 0)),
    )(lx)


def _post(e_agg, t_agg, ex, lx):
    spec = pl.BlockSpec((_BR, D), lambda i: (i, 0))
    return pl.pallas_call(
        _post_body,
        out_shape=(
            jax.ShapeDtypeStruct((N, D), jnp.float32),
            jax.ShapeDtypeStruct((N, D), jnp.float32),
        ),
        grid=(N // _BR,),
        in_specs=[spec, spec, spec, spec],
        out_specs=(spec, spec),
    )(e_agg, t_agg, ex, lx)


def _bcast_lane(v16, lane):
    idx = jnp.full((16, 1), lane, jnp.int32)
    dnums = lax.GatherDimensionNumbers(
        offset_dims=(), collapsed_slice_dims=(0,), start_index_map=(0,))
    return lax.gather(v16, idx, dnums, (1,),
                      mode=lax.GatherScatterMode.PROMISE_IN_BOUNDS)

def _sc_body(ex_hbm, tx_hbm, src_hbm, dst_hbm, vals_hbm,
             eo_hbm, to_hbm,
             sidx, didx, vvals, rows, tmp, acc, sem):
    c = lax.axis_index("c")
    s = lax.axis_index("s")

    z = jnp.zeros((16,), jnp.float32)
    for i in range(RCHUNK):
        for j in range(8):
            tmp[i, pl.ds(j * 16, 16)] = z

    def zbody(k, carry):
        blk = k * TILES + s

        @pl.when(blk < NRBLK)
        def _():
            r0 = pl.multiple_of(blk * RCHUNK, 8)
            pltpu.sync_copy(tmp, acc.at[pl.ds(r0, RCHUNK)])

        return carry

    lax.fori_loop(0, NRIT, zbody, 0)
    plsc.subcore_barrier()

    def run(x_hbm, out_hbm):
        def chunk_body(i, carry):
            base = pl.multiple_of(s * EPT + i * CHUNK, 8)
            pltpu.sync_copy(src_hbm.at[pl.ds(base, CHUNK)], sidx)
            pltpu.sync_copy(dst_hbm.at[pl.ds(base, CHUNK)], didx)
            pltpu.sync_copy(vals_hbm.at[pl.ds(base, CHUNK)], vvals)
            pltpu.async_copy(x_hbm.at[sidx], rows, sem).wait()

            def scale(g, carry2):
                v16 = vvals[pl.ds(g * 16, 16)]
                for lane in range(16):
                    vb = _bcast_lane(v16, lane)
                    e = g * 16 + lane
                    for j in range(8):
                        sl = pl.ds(j * 16, 16)
                        rows[e, sl] = rows[e, sl] * vb
                return carry2

            lax.fori_loop(0, CHUNK // 16, scale, 0)
            pltpu.sync_copy(rows, acc.at[didx], add=True)
            return carry

        lax.fori_loop(0, NCHUNK, chunk_body, 0)
        plsc.subcore_barrier()

        def rbody(k, carry):
            blk = k * TILES + s

            @pl.when(blk < NRBLK)
            def _():
                r0 = pl.multiple_of(blk * RCHUNK, 8)
                pltpu.sync_copy(acc.at[pl.ds(r0, RCHUNK)], tmp)
                pltpu.sync_copy(tmp, out_hbm.at[pl.ds(r0, RCHUNK)])

            return carry

        lax.fori_loop(0, NRIT, rbody, 0)

    @pl.when(c == 0)
    def _():
        run(ex_hbm, eo_hbm)

    @pl.when(c == 1)
    def _():
        run(tx_hbm, to_hbm)


@functools.partial(
    pl.kernel,
    out_type=(
        jax.ShapeDtypeStruct((N, D), jnp.float32),
        jax.ShapeDtypeStruct((N, D), jnp.float32),
    ),
    mesh=plsc.VectorSubcoreMesh(core_axis_name="c", subcore_axis_name="s"),
    scratch_types=[
        pltpu.VMEM((CHUNK,), jnp.int32),
        pltpu.VMEM((CHUNK,), jnp.int32),
        pltpu.VMEM((CHUNK,), jnp.float32),
        pltpu.VMEM((CHUNK, D), jnp.float32),
        pltpu.VMEM((RCHUNK, D), jnp.float32),
        pltpu.VMEM_SHARED((N, D), jnp.float32),
        pltpu.SemaphoreType.DMA,
    ],
)
def _sc_spmm(*refs):
    _sc_body(*refs)


def kernel(euclidean_x, lorentz_x, adj_indices, adj_values):
    dst = adj_indices[0]
    src = adj_indices[1]
    tangent_x = _pre(lorentz_x)
    e_agg, t_agg = _sc_spmm(euclidean_x, tangent_x, src, dst, adj_values)
    return _post(e_agg, t_agg, euclidean_x, lorentz_x)


# speedup vs baseline: 9.6743x; 2.4947x over previous
import functools

import jax
import jax.numpy as jnp
from jax import lax
from jax.experimental import pallas as pl
from jax.experimental.pallas import tpu as pltpu
from jax.experimental.pallas import tpu_sc as plsc

N = 10000
E = 320000
D = 128
EPS = 1e-7

TILES = 16
EPT = E // TILES
CHUNK = 80
NCHUNK = EPT // CHUNK
BLK = 50
NBLK = NCHUNK // BLK
RCHUNK = 40
NRBLK = N // RCHUNK
NRIT = -(-NRBLK // TILES)


def _norm(x):
    return jnp.maximum(jnp.sqrt(jnp.sum(x * x, axis=-1, keepdims=True)), EPS)


def _artanh(x):
    x = jnp.clip(x, -1.0 + 1e-6, 1.0 - 1e-6)
    return 0.5 * jnp.log((1.0 + x) / (1.0 - x))


def _pre_body(lx_ref, o_ref):
    x = lx_ref[...]
    n = _norm(x)
    o_ref[...] = _artanh(n) * x / n


def _post_body(eagg_ref, tagg_ref, ex_ref, lx_ref, eo_ref, lo_ref):
    ex = ex_ref[...]
    lx = lx_ref[...]
    eo_ref[...] = 0.5 * eagg_ref[...] + 0.5 * ex

    t = tagg_ref[...]
    nt = _norm(t)
    lorentz_out = jnp.tanh(nt) * t / nt

    def mobius_scalar_mul(r, x):
        n = _norm(x)
        return jnp.tanh(r * _artanh(n)) * x / n

    l_skip = mobius_scalar_mul(0.5, lx)
    l_out = mobius_scalar_mul(0.5, lorentz_out)

    xy = jnp.sum(l_out * l_skip, axis=-1, keepdims=True)
    x2 = jnp.sum(l_out * l_out, axis=-1, keepdims=True)
    y2 = jnp.sum(l_skip * l_skip, axis=-1, keepdims=True)
    num = (1.0 + 2.0 * xy + y2) * l_out + (1.0 - x2) * l_skip
    den = jnp.maximum(1.0 + 2.0 * xy + x2 * y2, EPS)
    lo_ref[...] = num / den


_BR = 1000


def _pre(lx):
    return pl.pallas_call(
        _pre_body,
        out_shape=jax.ShapeDtypeStruct((N, D), jnp.float32),
        grid=(N // _BR,),
        in_specs=[pl.BlockSpec((_BR, D), lambda i: (i, 0))],
        out_specs=pl.BlockSpec((_BR, D), lambda i: (i, 0)),
    )(lx)


def _post(e_agg, t_agg, ex, lx):
    spec = pl.BlockSpec((_BR, D), lambda i: (i, 0))
    return pl.pallas_call(
        _post_body,
        out_shape=(
            jax.ShapeDtypeStruct((N, D), jnp.float32),
            jax.ShapeDtypeStruct((N, D), jnp.float32),
        ),
        grid=(N // _BR,),
        in_specs=[spec, spec, spec, spec],
        out_specs=(spec, spec),
    )(e_agg, t_agg, ex, lx)


def _bcast_lane(v16, lane):
    idx = jnp.full((16, 1), lane, jnp.int32)
    dnums = lax.GatherDimensionNumbers(
        offset_dims=(), collapsed_slice_dims=(0,), start_index_map=(0,))
    return lax.gather(v16, idx, dnums, (1,),
                      mode=lax.GatherScatterMode.PROMISE_IN_BOUNDS)

def _sc_body(ex_hbm, tx_hbm, src_hbm, dst_hbm, vals_hbm,
             eo_hbm, to_hbm,
             sblk, dblk, vblk, rows_a, rows_b, acc,
             sem_a, sem_b):
    c = lax.axis_index("c")
    s = lax.axis_index("s")

    z = jnp.zeros((16,), jnp.float32)
    for i in range(RCHUNK):
        for j in range(8):
            rows_a[i, pl.ds(j * 16, 16)] = z

    def zbody(k, carry):
        blk = k * TILES + s

        @pl.when(blk < NRBLK)
        def _():
            r0 = pl.multiple_of(blk * RCHUNK, 8)
            pltpu.sync_copy(rows_a.at[pl.ds(0, RCHUNK)], acc.at[pl.ds(r0, RCHUNK)])

        return carry

    lax.fori_loop(0, NRIT, zbody, 0)

    def run(x_hbm, out_hbm):
        def process(i, rows):
            def scale(g, carry2):
                v16 = vblk[i, pl.ds(g * 16, 16)]
                for lane in range(16):
                    vb = _bcast_lane(v16, lane)
                    e = g * 16 + lane
                    for j in range(8):
                        sl = pl.ds(j * 16, 16)
                        rows[e, sl] = rows[e, sl] * vb
                return carry2

            lax.fori_loop(0, CHUNK // 16, scale, 0)
            pltpu.sync_copy(rows, acc.at[dblk.at[i]], add=True)

        def block_body(b, carry):
            pltpu.sync_copy(src_hbm.at[s, b], sblk)
            pltpu.sync_copy(dst_hbm.at[s, b], dblk)
            pltpu.sync_copy(vals_hbm.at[s, b], vblk)

            pltpu.async_copy(x_hbm.at[sblk.at[0]], rows_a, sem_a)

            def pair_body(p, carry2):
                i0 = 2 * p
                i1 = 2 * p + 1
                pltpu.async_copy(x_hbm.at[sblk.at[i1]], rows_b, sem_b)
                pltpu.make_async_copy(x_hbm.at[sblk.at[i0]], rows_a, sem_a).wait()
                process(i0, rows_a)

                @pl.when(i1 + 1 < BLK)
                def _():
                    pltpu.async_copy(x_hbm.at[sblk.at[i1 + 1]], rows_a, sem_a)

                pltpu.make_async_copy(x_hbm.at[sblk.at[i1]], rows_b, sem_b).wait()
                process(i1, rows_b)
                return carry2

            lax.fori_loop(0, BLK // 2, pair_body, 0)
            return carry

        lax.fori_loop(0, NBLK, block_body, 0)
        plsc.subcore_barrier()

        def rbody(k, carry):
            blk = k * TILES + s

            @pl.when(blk < NRBLK)
            def _():
                r0 = pl.multiple_of(blk * RCHUNK, 8)
                pltpu.sync_copy(acc.at[pl.ds(r0, RCHUNK)], rows_a.at[pl.ds(0, RCHUNK)])
                pltpu.sync_copy(rows_a.at[pl.ds(0, RCHUNK)], out_hbm.at[pl.ds(r0, RCHUNK)])

            return carry

        lax.fori_loop(0, NRIT, rbody, 0)

    plsc.subcore_barrier()

    @pl.when(c == 0)
    def _():
        run(ex_hbm, eo_hbm)

    @pl.when(c == 1)
    def _():
        run(tx_hbm, to_hbm)


@functools.partial(
    pl.kernel,
    out_type=(
        jax.ShapeDtypeStruct((N, D), jnp.float32),
        jax.ShapeDtypeStruct((N, D), jnp.float32),
    ),
    mesh=plsc.VectorSubcoreMesh(core_axis_name="c", subcore_axis_name="s"),
    scratch_types=[
        pltpu.VMEM((BLK, CHUNK), jnp.int32),
        pltpu.VMEM((BLK, CHUNK), jnp.int32),
        pltpu.VMEM((BLK, CHUNK), jnp.float32),
        pltpu.VMEM((CHUNK, D), jnp.float32),
        pltpu.VMEM((CHUNK, D), jnp.float32),
        pltpu.VMEM_SHARED((N, D), jnp.float32),
        pltpu.SemaphoreType.DMA,
        pltpu.SemaphoreType.DMA,
    ],
)
def _sc_spmm(*refs):
    _sc_body(*refs)


def kernel(euclidean_x, lorentz_x, adj_indices, adj_values):
    dst = adj_indices[0].reshape(TILES, NBLK, BLK, CHUNK)
    src = adj_indices[1].reshape(TILES, NBLK, BLK, CHUNK)
    vals = adj_values.reshape(TILES, NBLK, BLK, CHUNK)
    tangent_x = _pre(lorentz_x)
    e_agg, t_agg = _sc_spmm(euclidean_x, tangent_x, src, dst, vals)
    return _post(e_agg, t_agg, euclidean_x, lorentz_x)


# speedup vs baseline: 10.9397x; 1.1308x over previous
import functools

import jax
import jax.numpy as jnp
from jax import lax
from jax.experimental import pallas as pl
from jax.experimental.pallas import tpu as pltpu
from jax.experimental.pallas import tpu_sc as plsc

N = 10000
E = 320000
D = 128
EPS = 1e-7

TILES = 16
EPT = E // TILES
CHUNK = 80
NCHUNK = EPT // CHUNK
BLK = 50
NBLK = NCHUNK // BLK
RCHUNK = 40
NRBLK = N // RCHUNK
NRIT = -(-NRBLK // TILES)


def _norm(x):
    return jnp.maximum(jnp.sqrt(jnp.sum(x * x, axis=-1, keepdims=True)), EPS)


def _artanh(x):
    x = jnp.clip(x, -1.0 + 1e-6, 1.0 - 1e-6)
    return 0.5 * jnp.log((1.0 + x) / (1.0 - x))


def _pre_body(lx_ref, o_ref):
    x = lx_ref[...]
    n = _norm(x)
    o_ref[...] = _artanh(n) * x / n


def _post_body(eagg_ref, tagg_ref, ex_ref, lx_ref, eo_ref, lo_ref):
    ex = ex_ref[...]
    lx = lx_ref[...]
    eo_ref[...] = 0.5 * eagg_ref[...] + 0.5 * ex

    t = tagg_ref[...]
    nt = _norm(t)
    lorentz_out = jnp.tanh(nt) * t / nt

    def mobius_scalar_mul(r, x):
        n = _norm(x)
        return jnp.tanh(r * _artanh(n)) * x / n

    l_skip = mobius_scalar_mul(0.5, lx)
    l_out = mobius_scalar_mul(0.5, lorentz_out)

    xy = jnp.sum(l_out * l_skip, axis=-1, keepdims=True)
    x2 = jnp.sum(l_out * l_out, axis=-1, keepdims=True)
    y2 = jnp.sum(l_skip * l_skip, axis=-1, keepdims=True)
    num = (1.0 + 2.0 * xy + y2) * l_out + (1.0 - x2) * l_skip
    den = jnp.maximum(1.0 + 2.0 * xy + x2 * y2, EPS)
    lo_ref[...] = num / den


_BR = 1000


def _pre(lx):
    return pl.pallas_call(
        _pre_body,
        out_shape=jax.ShapeDtypeStruct((N, D), jnp.float32),
        grid=(N // _BR,),
        in_specs=[pl.BlockSpec((_BR, D), lambda i: (i, 0))],
        out_specs=pl.BlockSpec((_BR, D), lambda i: (i, 0)),
    )(lx)


def _post(e_agg, t_agg, ex, lx):
    spec = pl.BlockSpec((_BR, D), lambda i: (i, 0))
    return pl.pallas_call(
        _post_body,
        out_shape=(
            jax.ShapeDtypeStruct((N, D), jnp.float32),
            jax.ShapeDtypeStruct((N, D), jnp.float32),
        ),
        grid=(N // _BR,),
        in_specs=[spec, spec, spec, spec],
        out_specs=(spec, spec),
    )(e_agg, t_agg, ex, lx)


def _bcast_lane(v16, lane):
    idx = jnp.full((16, 1), lane, jnp.int32)
    dnums = lax.GatherDimensionNumbers(
        offset_dims=(), collapsed_slice_dims=(0,), start_index_map=(0,))
    return lax.gather(v16, idx, dnums, (1,),
                      mode=lax.GatherScatterMode.PROMISE_IN_BOUNDS)

def _sc_body(ex_hbm, tx_hbm, pk_hbm, vals_hbm,
             eo_hbm, to_hbm,
             pblk, vblk, si_a, si_b, si_c, di_a, di_b, di_c,
             rows_a, rows_b, rows_c, acc,
             gs_a, gs_b, gs_c, ss_a, ss_b, ss_c):
    c = lax.axis_index("c")
    s = lax.axis_index("s")

    z = jnp.zeros((16,), jnp.float32)
    for i in range(RCHUNK):
        for j in range(8):
            rows_a[i, pl.ds(j * 16, 16)] = z

    def zbody(k, carry):
        blk = k * TILES + s

        @pl.when(blk < NRBLK)
        def _():
            r0 = pl.multiple_of(blk * RCHUNK, 8)
            pltpu.sync_copy(rows_a.at[pl.ds(0, RCHUNK)], acc.at[pl.ds(r0, RCHUNK)])

        return carry

    lax.fori_loop(0, NRIT, zbody, 0)

    def run(x_hbm, out_hbm):
        bufs = (rows_a, rows_b, rows_c)
        sidx = (si_a, si_b, si_c)
        didx = (di_a, di_b, di_c)
        gsems = (gs_a, gs_b, gs_c)
        ssems = (ss_a, ss_b, ss_c)

        def decode(i, b):
            for q in range(CHUNK // 16):
                sl = pl.ds(q * 16, 16)
                pk = pblk[i, sl]
                sidx[b][sl] = lax.shift_right_logical(pk, 14)
                didx[b][sl] = lax.bitwise_and(pk, (1 << 14) - 1)

        def g(i, b):
            pltpu.async_copy(x_hbm.at[sidx[b]], bufs[b], gsems[b])

        def gwait(i, b):
            pltpu.make_async_copy(x_hbm.at[sidx[b]], bufs[b], gsems[b]).wait()

        def scat(i, b):
            pltpu.async_copy(bufs[b], acc.at[didx[b]], ssems[b], add=True)

        def swait(i, b):
            pltpu.make_async_copy(bufs[b], acc.at[didx[b]], ssems[b]).wait()

        def scale(i, rows):
            def grp(gi, carry2):
                v16 = vblk[i, pl.ds(gi * 16, 16)]
                for lane in range(16):
                    vb = _bcast_lane(v16, lane)
                    e = gi * 16 + lane
                    for j in range(8):
                        sl = pl.ds(j * 16, 16)
                        rows[e, sl] = rows[e, sl] * vb
                return carry2

            lax.fori_loop(0, CHUNK // 16, grp, 0)

        def block_body(bk, carry):
            pltpu.sync_copy(pk_hbm.at[s, bk], pblk)
            pltpu.sync_copy(vals_hbm.at[s, bk], vblk)

            decode(0, 0)
            g(0, 0)
            decode(1, 1)
            g(1, 1)

            def tri(t, carry2):
                i = 3 * t
                gwait(i, 0)
                scale(i, rows_a)
                scat(i, 0)

                @pl.when(t > 0)
                def _():
                    swait(i - 1, 2)

                decode(i + 2, 2)
                g(i + 2, 2)
                gwait(i + 1, 1)
                scale(i + 1, rows_b)
                scat(i + 1, 1)
                swait(i, 0)
                decode(i + 3, 0)
                g(i + 3, 0)
                gwait(i + 2, 2)
                scale(i + 2, rows_c)
                scat(i + 2, 2)
                swait(i + 1, 1)
                decode(i + 4, 1)
                g(i + 4, 1)
                return carry2

            lax.fori_loop(0, (BLK - 2) // 3, tri, 0)
            gwait(BLK - 2, 0)
            scale(BLK - 2, rows_a)
            scat(BLK - 2, 0)
            gwait(BLK - 1, 1)
            scale(BLK - 1, rows_b)
            scat(BLK - 1, 1)
            swait(BLK - 3, 2)
            swait(BLK - 2, 0)
            swait(BLK - 1, 1)
            return carry

        lax.fori_loop(0, NBLK, block_body, 0)
        plsc.subcore_barrier()

        def rbody(k, carry):
            blk = k * TILES + s

            @pl.when(blk < NRBLK)
            def _():
                r0 = pl.multiple_of(blk * RCHUNK, 8)
                pltpu.sync_copy(acc.at[pl.ds(r0, RCHUNK)], rows_a.at[pl.ds(0, RCHUNK)])
                pltpu.sync_copy(rows_a.at[pl.ds(0, RCHUNK)], out_hbm.at[pl.ds(r0, RCHUNK)])

            return carry

        lax.fori_loop(0, NRIT, rbody, 0)

    plsc.subcore_barrier()

    @pl.when(c == 0)
    def _():
        run(ex_hbm, eo_hbm)

    @pl.when(c == 1)
    def _():
        run(tx_hbm, to_hbm)


@functools.partial(
    pl.kernel,
    out_type=(
        jax.ShapeDtypeStruct((N, D), jnp.float32),
        jax.ShapeDtypeStruct((N, D), jnp.float32),
    ),
    mesh=plsc.VectorSubcoreMesh(core_axis_name="c", subcore_axis_name="s"),
    scratch_types=[
        pltpu.VMEM((BLK, CHUNK), jnp.int32),
        pltpu.VMEM((BLK, CHUNK), jnp.float32),
        pltpu.VMEM((CHUNK,), jnp.int32),
        pltpu.VMEM((CHUNK,), jnp.int32),
        pltpu.VMEM((CHUNK,), jnp.int32),
        pltpu.VMEM((CHUNK,), jnp.int32),
        pltpu.VMEM((CHUNK,), jnp.int32),
        pltpu.VMEM((CHUNK,), jnp.int32),
        pltpu.VMEM((CHUNK, D), jnp.float32),
        pltpu.VMEM((CHUNK, D), jnp.float32),
        pltpu.VMEM((CHUNK, D), jnp.float32),
        pltpu.VMEM_SHARED((N, D), jnp.float32),
        pltpu.SemaphoreType.DMA,
        pltpu.SemaphoreType.DMA,
        pltpu.SemaphoreType.DMA,
        pltpu.SemaphoreType.DMA,
        pltpu.SemaphoreType.DMA,
        pltpu.SemaphoreType.DMA,
    ],
)
def _sc_spmm(*refs):
    _sc_body(*refs)


def kernel(euclidean_x, lorentz_x, adj_indices, adj_values):
    packed = (adj_indices[1] * 16384 + adj_indices[0]).reshape(
        TILES, NBLK, BLK, CHUNK)
    vals = adj_values.reshape(TILES, NBLK, BLK, CHUNK)
    tangent_x = _pre(lorentz_x)
    e_agg, t_agg = _sc_spmm(euclidean_x, tangent_x, packed, vals)
    return _post(e_agg, t_agg, euclidean_x, lorentz_x)


# speedup vs baseline: 10.9723x; 1.0030x over previous
import functools

import jax
import jax.numpy as jnp
from jax import lax
from jax.experimental import pallas as pl
from jax.experimental.pallas import tpu as pltpu
from jax.experimental.pallas import tpu_sc as plsc

N = 10000
E = 320000
D = 128
EPS = 1e-7

TILES = 16
EPT = E // TILES
CHUNK = 80
NCHUNK = EPT // CHUNK
BLK = 50
NBLK = NCHUNK // BLK
RCHUNK = 40
NRBLK = N // RCHUNK
NRIT = -(-NRBLK // TILES)


def _norm(x):
    return jnp.maximum(jnp.sqrt(jnp.sum(x * x, axis=-1, keepdims=True)), EPS)


def _artanh(x):
    x = jnp.clip(x, -1.0 + 1e-6, 1.0 - 1e-6)
    return 0.5 * jnp.log((1.0 + x) / (1.0 - x))


def _pre_body(lx_ref, o_ref):
    x = lx_ref[...]
    n = _norm(x)
    o_ref[...] = _artanh(n) * x / n


def _post_body(eagg_ref, tagg_ref, ex_ref, lx_ref, eo_ref, lo_ref):
    ex = ex_ref[...]
    lx = lx_ref[...]
    eo_ref[...] = 0.5 * eagg_ref[...] + 0.5 * ex

    t = tagg_ref[...]
    nt = _norm(t)
    lorentz_out = jnp.tanh(nt) * t / nt

    def mobius_scalar_mul(r, x):
        n = _norm(x)
        return jnp.tanh(r * _artanh(n)) * x / n

    l_skip = mobius_scalar_mul(0.5, lx)
    l_out = mobius_scalar_mul(0.5, lorentz_out)

    xy = jnp.sum(l_out * l_skip, axis=-1, keepdims=True)
    x2 = jnp.sum(l_out * l_out, axis=-1, keepdims=True)
    y2 = jnp.sum(l_skip * l_skip, axis=-1, keepdims=True)
    num = (1.0 + 2.0 * xy + y2) * l_out + (1.0 - x2) * l_skip
    den = jnp.maximum(1.0 + 2.0 * xy + x2 * y2, EPS)
    lo_ref[...] = num / den


_BR = 2000


def _pre(lx):
    return pl.pallas_call(
        _pre_body,
        out_shape=jax.ShapeDtypeStruct((N, D), jnp.float32),
        grid=(N // _BR,),
        in_specs=[pl.BlockSpec((_BR, D), lambda i: (i, 0))],
        out_specs=pl.BlockSpec((_BR, D), lambda i: (i, 0)),
    )(lx)


def _post(e_agg, t_agg, ex, lx):
    spec = pl.BlockSpec((_BR, D), lambda i: (i, 0))
    return pl.pallas_call(
        _post_body,
        out_shape=(
            jax.ShapeDtypeStruct((N, D), jnp.float32),
            jax.ShapeDtypeStruct((N, D), jnp.float32),
        ),
        grid=(N // _BR,),
        in_specs=[spec, spec, spec, spec],
        out_specs=(spec, spec),
    )(e_agg, t_agg, ex, lx)


def _bcast_lane(v16, lane):
    idx = jnp.full((16, 1), lane, jnp.int32)
    dnums = lax.GatherDimensionNumbers(
        offset_dims=(), collapsed_slice_dims=(0,), start_index_map=(0,))
    return lax.gather(v16, idx, dnums, (1,),
                      mode=lax.GatherScatterMode.PROMISE_IN_BOUNDS)

def _sc_body(ex_hbm, tx_hbm, pk_hbm, vals_hbm,
             eo_hbm, to_hbm,
             pblk, vblk, si_a, si_b, si_c, di_a, di_b, di_c,
             rows_a, rows_b, rows_c, acc,
             gs_a, gs_b, gs_c, ss_a, ss_b, ss_c):
    c = lax.axis_index("c")
    s = lax.axis_index("s")

    z = jnp.zeros((16,), jnp.float32)
    for i in range(RCHUNK):
        for j in range(8):
            rows_a[i, pl.ds(j * 16, 16)] = z

    def zbody(k, carry):
        blk = k * TILES + s

        @pl.when(blk < NRBLK)
        def _():
            r0 = pl.multiple_of(blk * RCHUNK, 8)
            pltpu.sync_copy(rows_a.at[pl.ds(0, RCHUNK)], acc.at[pl.ds(r0, RCHUNK)])

        return carry

    lax.fori_loop(0, NRIT, zbody, 0)

    def run(x_hbm, out_hbm):
        bufs = (rows_a, rows_b, rows_c)
        sidx = (si_a, si_b, si_c)
        didx = (di_a, di_b, di_c)
        gsems = (gs_a, gs_b, gs_c)
        ssems = (ss_a, ss_b, ss_c)

        def decode(i, b):
            for q in range(CHUNK // 16):
                sl = pl.ds(q * 16, 16)
                pk = pblk[i, sl]
                sidx[b][sl] = lax.shift_right_logical(pk, 14)
                didx[b][sl] = lax.bitwise_and(pk, (1 << 14) - 1)

        def g(i, b):
            pltpu.async_copy(x_hbm.at[sidx[b]], bufs[b], gsems[b])

        def gwait(i, b):
            pltpu.make_async_copy(x_hbm.at[sidx[b]], bufs[b], gsems[b]).wait()

        def scat(i, b):
            pltpu.async_copy(bufs[b], acc.at[didx[b]], ssems[b], add=True)

        def swait(i, b):
            pltpu.make_async_copy(bufs[b], acc.at[didx[b]], ssems[b]).wait()

        def scale(i, rows):
            def grp(gi, carry2):
                v16 = vblk[i, pl.ds(gi * 16, 16)]
                for lane in range(16):
                    vb = _bcast_lane(v16, lane)
                    e = gi * 16 + lane
                    for j in range(8):
                        sl = pl.ds(j * 16, 16)
                        rows[e, sl] = rows[e, sl] * vb
                return carry2

            lax.fori_loop(0, CHUNK // 16, grp, 0)

        def block_body(bk, carry):
            pltpu.sync_copy(pk_hbm.at[s, bk], pblk)
            pltpu.sync_copy(vals_hbm.at[s, bk], vblk)

            decode(0, 0)
            g(0, 0)
            decode(1, 1)
            g(1, 1)

            def tri(t, carry2):
                i = 3 * t
                gwait(i, 0)
                scale(i, rows_a)
                scat(i, 0)

                @pl.when(t > 0)
                def _():
                    swait(i - 1, 2)

                decode(i + 2, 2)
                g(i + 2, 2)
                gwait(i + 1, 1)
                scale(i + 1, rows_b)
                scat(i + 1, 1)
                swait(i, 0)
                decode(i + 3, 0)
                g(i + 3, 0)
                gwait(i + 2, 2)
                scale(i + 2, rows_c)
                scat(i + 2, 2)
                swait(i + 1, 1)
                decode(i + 4, 1)
                g(i + 4, 1)
                return carry2

            lax.fori_loop(0, (BLK - 2) // 3, tri, 0)
            gwait(BLK - 2, 0)
            scale(BLK - 2, rows_a)
            scat(BLK - 2, 0)
            gwait(BLK - 1, 1)
            scale(BLK - 1, rows_b)
            scat(BLK - 1, 1)
            swait(BLK - 3, 2)
            swait(BLK - 2, 0)
            swait(BLK - 1, 1)
            return carry

        lax.fori_loop(0, NBLK, block_body, 0)
        plsc.subcore_barrier()

        def rbody(k, carry):
            blk = k * TILES + s

            @pl.when(blk < NRBLK)
            def _():
                r0 = pl.multiple_of(blk * RCHUNK, 8)
                pltpu.sync_copy(acc.at[pl.ds(r0, RCHUNK)], out_hbm.at[pl.ds(r0, RCHUNK)])

            return carry

        lax.fori_loop(0, NRIT, rbody, 0)

    plsc.subcore_barrier()

    @pl.when(c == 0)
    def _():
        run(ex_hbm, eo_hbm)

    @pl.when(c == 1)
    def _():
        run(tx_hbm, to_hbm)


@functools.partial(
    pl.kernel,
    out_type=(
        jax.ShapeDtypeStruct((N, D), jnp.float32),
        jax.ShapeDtypeStruct((N, D), jnp.float32),
    ),
    mesh=plsc.VectorSubcoreMesh(core_axis_name="c", subcore_axis_name="s"),
    scratch_types=[
        pltpu.VMEM((BLK, CHUNK), jnp.int32),
        pltpu.VMEM((BLK, CHUNK), jnp.float32),
        pltpu.VMEM((CHUNK,), jnp.int32),
        pltpu.VMEM((CHUNK,), jnp.int32),
        pltpu.VMEM((CHUNK,), jnp.int32),
        pltpu.VMEM((CHUNK,), jnp.int32),
        pltpu.VMEM((CHUNK,), jnp.int32),
        pltpu.VMEM((CHUNK,), jnp.int32),
        pltpu.VMEM((CHUNK, D), jnp.float32),
        pltpu.VMEM((CHUNK, D), jnp.float32),
        pltpu.VMEM((CHUNK, D), jnp.float32),
        pltpu.VMEM_SHARED((N, D), jnp.float32),
        pltpu.SemaphoreType.DMA,
        pltpu.SemaphoreType.DMA,
        pltpu.SemaphoreType.DMA,
        pltpu.SemaphoreType.DMA,
        pltpu.SemaphoreType.DMA,
        pltpu.SemaphoreType.DMA,
    ],
)
def _sc_spmm(*refs):
    _sc_body(*refs)


def kernel(euclidean_x, lorentz_x, adj_indices, adj_values):
    packed = (adj_indices[1] * 16384 + adj_indices[0]).reshape(
        TILES, NBLK, BLK, CHUNK)
    vals = adj_values.reshape(TILES, NBLK, BLK, CHUNK)
    tangent_x = _pre(lorentz_x)
    e_agg, t_agg = _sc_spmm(euclidean_x, tangent_x, packed, vals)
    return _post(e_agg, t_agg, euclidean_x, lorentz_x)
